# Initial kernel scaffold; baseline (speedup 1.0000x reference)
import functools

import jax
import jax.numpy as jnp
from jax import lax
from jax.experimental import pallas as pl
from jax.experimental.pallas import tpu as pltpu
from jax.experimental.pallas import tpu_sc as plsc

_NUM_CORES = 2
_NUM_SUBCORES = 16
_NUM_WORKERS = _NUM_CORES * _NUM_SUBCORES
_CHUNK = 128


@functools.lru_cache(maxsize=None)
def _make_gather(n_rows, row_len, batch):
    del n_rows
    b_per_w = batch // _NUM_WORKERS
    n_chunks = b_per_w // _CHUNK
    mesh = plsc.VectorSubcoreMesh(core_axis_name="c", subcore_axis_name="s")

    @functools.partial(
        pl.kernel,
        mesh=mesh,
        out_type=jax.ShapeDtypeStruct((batch, row_len), jnp.float32),
        scratch_types=[
            pltpu.VMEM((n_chunks, _CHUNK), jnp.int32),
            pltpu.VMEM((b_per_w, row_len), jnp.float32),
            pltpu.SemaphoreType.DMA,
        ],
    )
    def gather_kernel(ii_hbm, table_hbm, out_hbm, idx_v, rows_v, sem):
        wid = lax.axis_index("s") * _NUM_CORES + lax.axis_index("c")
        base = wid * b_per_w
        for j in range(n_chunks):
            pltpu.sync_copy(ii_hbm.at[pl.ds(base + j * _CHUNK, _CHUNK)],
                            idx_v.at[j])
        copies = []
        for j in range(n_chunks):
            copies.append(
                pltpu.async_copy(table_hbm.at[idx_v.at[j]],
                                 rows_v.at[pl.ds(j * _CHUNK, _CHUNK)], sem))
        for c in copies:
            c.wait()
        pltpu.sync_copy(rows_v, out_hbm.at[pl.ds(base, b_per_w)])

    return gather_kernel


@functools.lru_cache(maxsize=None)
def _make_matmul(batch, k_dim, d_out, blk):
    def mm_kernel(v_ref, w_ref, o_ref):
        o_ref[...] = jax.lax.dot_general(
            v_ref[...], w_ref[...], (((1,), (0,)), ((), ())),
            preferred_element_type=jnp.float32)

    return pl.pallas_call(
        mm_kernel,
        grid=(batch // blk,),
        in_specs=[
            pl.BlockSpec((blk, k_dim), lambda i: (i, 0)),
            pl.BlockSpec((k_dim, d_out), lambda i: (0, 0)),
        ],
        out_specs=pl.BlockSpec((blk, d_out), lambda i: (i, 0)),
        out_shape=jax.ShapeDtypeStruct((batch, d_out), jnp.float32),
    )


def kernel(ii, V, U):
    n_rows, d, n = V.shape
    _, d_out, _ = U.shape
    batch = ii.shape[0]
    row_len = d * n

    table = V.reshape(n_rows, row_len)
    idx = ii.astype(jnp.int32)

    v_flat = _make_gather(n_rows, row_len, batch)(idx, table)

    w_mat = jnp.transpose(U, (2, 0, 1)).reshape(row_len, d_out)
    x_ = _make_matmul(batch, row_len, d_out, 2048)(v_flat, w_mat)

    return x_, v_flat.reshape(batch, d, n)


# speedup vs baseline: 1.3667x; 1.3667x over previous
import functools

import jax
import jax.numpy as jnp
from jax import lax
from jax.experimental import pallas as pl
from jax.experimental.pallas import tpu as pltpu
from jax.experimental.pallas import tpu_sc as plsc

_NUM_CORES = 2
_NUM_SUBCORES = 16
_NUM_WORKERS = _NUM_CORES * _NUM_SUBCORES
_CHUNK = 128


@functools.lru_cache(maxsize=None)
def _make_gather(n_rows, row_len, batch):
    del n_rows
    b_per_w = batch // _NUM_WORKERS
    n_chunks = b_per_w // _CHUNK
    mesh = plsc.VectorSubcoreMesh(core_axis_name="c", subcore_axis_name="s")

    @functools.partial(
        pl.kernel,
        mesh=mesh,
        out_type=jax.ShapeDtypeStruct((batch, row_len), jnp.float32),
        compiler_params=pltpu.CompilerParams(use_tc_tiling_on_sc=False),
        scratch_types=[
            pltpu.VMEM((n_chunks, _CHUNK), jnp.int32),
            pltpu.VMEM((b_per_w, row_len), jnp.float32),
            pltpu.SemaphoreType.DMA,
        ],
    )
    def gather_kernel(ii_hbm, table_hbm, out_hbm, idx_v, rows_v, sem):
        wid = lax.axis_index("s") * _NUM_CORES + lax.axis_index("c")
        base = wid * b_per_w
        for j in range(n_chunks):
            pltpu.sync_copy(ii_hbm.at[pl.ds(base + j * _CHUNK, _CHUNK)],
                            idx_v.at[j])
        copies = []
        for j in range(n_chunks):
            copies.append(
                pltpu.async_copy(table_hbm.at[idx_v.at[j]],
                                 rows_v.at[pl.ds(j * _CHUNK, _CHUNK)], sem))
        for c in copies:
            c.wait()
        pltpu.sync_copy(rows_v, out_hbm.at[pl.ds(base, b_per_w)])

    return gather_kernel


@functools.lru_cache(maxsize=None)
def _make_matmul(batch, k_dim, d_out, blk):
    def mm_kernel(v_ref, w_ref, o_ref):
        o_ref[...] = jax.lax.dot_general(
            v_ref[...], w_ref[...], (((1,), (0,)), ((), ())),
            preferred_element_type=jnp.float32)

    return pl.pallas_call(
        mm_kernel,
        grid=(batch // blk,),
        in_specs=[
            pl.BlockSpec((blk, k_dim), lambda i: (i, 0)),
            pl.BlockSpec((k_dim, d_out), lambda i: (0, 0)),
        ],
        out_specs=pl.BlockSpec((blk, d_out), lambda i: (i, 0)),
        out_shape=jax.ShapeDtypeStruct((batch, d_out), jnp.float32),
    )


def kernel(ii, V, U):
    n_rows, d, n = V.shape
    _, d_out, _ = U.shape
    batch = ii.shape[0]
    row_len = d * n

    table = V.reshape(n_rows, row_len)
    idx = ii.astype(jnp.int32)

    v_flat = _make_gather(n_rows, row_len, batch)(idx, table)

    w_mat = jnp.transpose(U, (2, 0, 1)).reshape(row_len, d_out)
    x_ = _make_matmul(batch, row_len, d_out, 2048)(v_flat, w_mat)

    return x_, v_flat.reshape(batch, d, n)


# speedup vs baseline: 9.3135x; 6.8147x over previous
import functools

import jax
import jax.numpy as jnp
from jax import lax
from jax.experimental import pallas as pl
from jax.experimental.pallas import tpu as pltpu
from jax.experimental.pallas import tpu_sc as plsc

_NUM_CORES = 2
_NUM_SUBCORES = 16
_NUM_WORKERS = _NUM_CORES * _NUM_SUBCORES
_OUT_CHUNK = 8192


@functools.lru_cache(maxsize=None)
def _make_gather_t(n_rows, n_cols, batch):
    rows_per_w = n_rows // _NUM_WORKERS
    n_out_chunks = batch // _OUT_CHUNK
    mesh = plsc.VectorSubcoreMesh(core_axis_name="c", subcore_axis_name="s")

    @functools.partial(
        pl.kernel,
        mesh=mesh,
        out_type=jax.ShapeDtypeStruct((n_rows, batch), jnp.float32),
        compiler_params=pltpu.CompilerParams(needs_layout_passes=False),
        scratch_types=[
            pltpu.VMEM((batch,), jnp.int32),
            pltpu.VMEM((n_cols,), jnp.float32),
            pltpu.VMEM((_OUT_CHUNK,), jnp.float32),
        ],
    )
    def gather_kernel(ii_hbm, table_hbm, out_hbm, idx_v, row_v, out_v):
        wid = lax.axis_index("s") * _NUM_CORES + lax.axis_index("c")
        base_r = wid * rows_per_w
        pltpu.sync_copy(ii_hbm, idx_v)
        for r_off in range(rows_per_w):
            r = base_r + r_off
            pltpu.sync_copy(table_hbm.at[r], row_v)
            for co in range(n_out_chunks):

                def body(i, _, co=co):
                    iv = idx_v[pl.ds(co * _OUT_CHUNK + i * 16, 16)]
                    out_v[pl.ds(i * 16, 16)] = plsc.load_gather(row_v, [iv])
                    return ()

                lax.fori_loop(0, _OUT_CHUNK // 16, body, (), unroll=4)
                pltpu.sync_copy(out_v,
                                out_hbm.at[r, pl.ds(co * _OUT_CHUNK,
                                                    _OUT_CHUNK)])

    return gather_kernel


@functools.lru_cache(maxsize=None)
def _make_matmul(batch, k_dim, d_out, blk):
    def mm_kernel(vt_ref, w_ref, o_ref):
        o_ref[...] = jax.lax.dot_general(
            vt_ref[...], w_ref[...], (((0,), (0,)), ((), ())),
            preferred_element_type=jnp.float32)

    return pl.pallas_call(
        mm_kernel,
        grid=(batch // blk,),
        in_specs=[
            pl.BlockSpec((k_dim, blk), lambda i: (0, i)),
            pl.BlockSpec((k_dim, d_out), lambda i: (0, 0)),
        ],
        out_specs=pl.BlockSpec((blk, d_out), lambda i: (i, 0)),
        out_shape=jax.ShapeDtypeStruct((batch, d_out), jnp.float32),
    )


def kernel(ii, V, U):
    n_cols, d, n = V.shape
    _, d_out, _ = U.shape
    batch = ii.shape[0]
    n_rows = d * n

    table_t = V.transpose(2, 1, 0).reshape(n_rows, n_cols)
    idx = ii.astype(jnp.int32)

    v_t = _make_gather_t(n_rows, n_cols, batch)(idx, table_t)

    u_mat = U.transpose(0, 2, 1).reshape(n_rows, d_out)
    x_ = _make_matmul(batch, n_rows, d_out, 2048)(v_t, u_mat)

    v = v_t.reshape(n, d, batch).transpose(2, 1, 0)
    return x_, v


# speedup vs baseline: 14.7798x; 1.5869x over previous
import functools

import jax
import jax.numpy as jnp
from jax import lax
from jax.experimental import pallas as pl
from jax.experimental.pallas import tpu as pltpu
from jax.experimental.pallas import tpu_sc as plsc

_NUM_CORES = 2
_NUM_SUBCORES = 16
_NUM_WORKERS = _NUM_CORES * _NUM_SUBCORES
_OUT_CHUNK = 4096


@functools.lru_cache(maxsize=None)
def _make_gather_t(n_rows, n_cols, batch):
    rows_per_w = n_rows // _NUM_WORKERS
    n_out_chunks = batch // _OUT_CHUNK
    mesh = plsc.VectorSubcoreMesh(core_axis_name="c", subcore_axis_name="s")

    @functools.partial(
        pl.kernel,
        mesh=mesh,
        out_type=jax.ShapeDtypeStruct((n_rows, batch), jnp.float32),
        compiler_params=pltpu.CompilerParams(needs_layout_passes=False),
        scratch_types=[
            pltpu.VMEM((batch,), jnp.int32),
            pltpu.VMEM((n_cols,), jnp.float32),
            pltpu.VMEM((2 * _OUT_CHUNK,), jnp.float32),
            pltpu.SemaphoreType.DMA,
        ],
    )
    def gather_kernel(ii_hbm, table_hbm, out_hbm, idx_v, row_v, out_v, sem):
        wid = lax.axis_index("s") * _NUM_CORES + lax.axis_index("c")
        base_r = wid * rows_per_w
        pltpu.sync_copy(ii_hbm, idx_v)
        copies = []
        for r_off in range(rows_per_w):
            r = base_r + r_off
            pltpu.sync_copy(table_hbm.at[r], row_v)
            for co in range(n_out_chunks):
                k = r_off * n_out_chunks + co
                if k >= 2:
                    copies[k - 2].wait()
                boff = (k % 2) * _OUT_CHUNK

                @plsc.parallel_loop(0, _OUT_CHUNK, step=16, unroll=8)
                def body(i, co=co, boff=boff):
                    iv = idx_v[pl.ds(co * _OUT_CHUNK + i, 16)]
                    out_v[pl.ds(boff + i, 16)] = plsc.load_gather(
                        row_v, [iv])

                copies.append(
                    pltpu.async_copy(
                        out_v.at[pl.ds(boff, _OUT_CHUNK)],
                        out_hbm.at[r, pl.ds(co * _OUT_CHUNK, _OUT_CHUNK)],
                        sem))
        for c in copies[-2:]:
            c.wait()

    return gather_kernel


@functools.lru_cache(maxsize=None)
def _make_matmul(batch, k_dim, d_out, blk):
    def mm_kernel(vt_ref, w_ref, o_ref):
        o_ref[...] = jax.lax.dot_general(
            vt_ref[...], w_ref[...], (((0,), (0,)), ((), ())),
            preferred_element_type=jnp.float32)

    return pl.pallas_call(
        mm_kernel,
        grid=(batch // blk,),
        in_specs=[
            pl.BlockSpec((k_dim, blk), lambda i: (0, i)),
            pl.BlockSpec((k_dim, d_out), lambda i: (0, 0)),
        ],
        out_specs=pl.BlockSpec((blk, d_out), lambda i: (i, 0)),
        out_shape=jax.ShapeDtypeStruct((batch, d_out), jnp.float32),
    )


def kernel(ii, V, U):
    n_cols, d, n = V.shape
    _, d_out, _ = U.shape
    batch = ii.shape[0]
    n_rows = d * n

    table_t = V.transpose(2, 1, 0).reshape(n_rows, n_cols)
    idx = ii.astype(jnp.int32)

    v_t = _make_gather_t(n_rows, n_cols, batch)(idx, table_t)

    u_mat = U.transpose(0, 2, 1).reshape(n_rows, d_out)
    x_ = _make_matmul(batch, n_rows, d_out, 2048)(v_t, u_mat)

    v = v_t.reshape(n, d, batch).transpose(2, 1, 0)
    return x_, v
